# baseline (speedup 1.0000x reference)
import functools

import jax
import jax.numpy as jnp
from jax import lax
from jax.experimental import pallas as pl
from jax.experimental.pallas import tpu as pltpu
from jax.experimental.pallas import tpu_sc as plsc

KMER_NUM = 1000000
DIM = 2
LATENT_DIM = 7
BATCH = 16384

NUM_CORES = 2
NUM_SUBCORES = 16
NUM_WORKERS = NUM_CORES * NUM_SUBCORES
PAIRS_PER_W = BATCH // NUM_WORKERS
LANES = 16
CHUNKS = PAIRS_PER_W // LANES

TC_BLOCK = 524288
TC_GRID = -(-KMER_NUM // TC_BLOCK)
F_ROWS = TC_GRID * TC_BLOCK // 128

LOG2E = 1.4426950408889634


def _tc_project_body(at_ref, et_ref, f0_ref, f1_ref):
    at = at_ref[...]
    for _ in range(2):
        at = jnp.exp(at - jnp.max(at, axis=1, keepdims=True))
        at = at / jnp.sum(at, axis=1, keepdims=True)
    e = et_ref[...]
    eb = (e * LOG2E).astype(jnp.bfloat16)
    ex = jnp.exp2(eb)
    mt = jnp.concatenate([at, jnp.ones((1, LATENT_DIM), jnp.float32)], axis=0)
    f3 = jax.lax.dot_general(
        mt.astype(jnp.bfloat16),
        ex,
        (((1,), (0,)), ((), ())),
        preferred_element_type=jnp.float32,
    )
    u0 = f3[0:1].reshape(TC_BLOCK // 128, 128)
    u1 = f3[1:2].reshape(TC_BLOCK // 128, 128)
    s = f3[2:3].reshape(TC_BLOCK // 128, 128)
    r = 1.0 / s
    f0_ref[...] = u0 * r
    f1_ref[...] = u1 * r


def _tc_project(At, embsT):
    return pl.pallas_call(
        _tc_project_body,
        grid=(TC_GRID,),
        in_specs=[
            pl.BlockSpec((DIM, LATENT_DIM), lambda i: (0, 0)),
            pl.BlockSpec((LATENT_DIM, TC_BLOCK), lambda i: (0, i)),
        ],
        out_specs=[
            pl.BlockSpec((TC_BLOCK // 128, 128), lambda i: (i, 0)),
            pl.BlockSpec((TC_BLOCK // 128, 128), lambda i: (i, 0)),
        ],
        out_shape=[
            jax.ShapeDtypeStruct((F_ROWS, 128), jnp.float32),
            jax.ShapeDtypeStruct((F_ROWS, 128), jnp.float32),
        ],
    )(At, embsT)


XBLK = PAIRS_PER_W // 128


def _sc_pair_loss(f0, f1, x3, degrees):
    mesh = plsc.VectorSubcoreMesh(core_axis_name="c", subcore_axis_name="s")

    @functools.partial(
        pl.kernel,
        mesh=mesh,
        compiler_params=pltpu.CompilerParams(use_tc_tiling_on_sc=False),
        out_type=jax.ShapeDtypeStruct((NUM_WORKERS * LANES,), jnp.float32),
        scratch_types=[
            pltpu.VMEM((PAIRS_PER_W,), jnp.int32),
            pltpu.VMEM((PAIRS_PER_W,), jnp.int32),
            pltpu.VMEM((PAIRS_PER_W,), jnp.float32),
            pltpu.VMEM((PAIRS_PER_W,), jnp.float32),
            pltpu.VMEM((PAIRS_PER_W,), jnp.float32),
            pltpu.VMEM((PAIRS_PER_W,), jnp.float32),
            pltpu.VMEM((PAIRS_PER_W,), jnp.float32),
            pltpu.VMEM((LANES,), jnp.float32),
            pltpu.SemaphoreType.DMA,
            pltpu.SemaphoreType.DMA,
            pltpu.SemaphoreType.DMA,
            pltpu.SemaphoreType.DMA,
        ],
    )
    def k(f0_hbm, f1_hbm, x3_hbm, deg_hbm, out_hbm,
          i0_v, i1_v, a0_v, a1_v, b0_v, b1_v, deg_v, acc_v,
          sem0, sem1, sem2, sem3):
        wid = lax.axis_index("s") * NUM_CORES + lax.axis_index("c")
        base = wid * PAIRS_PER_W
        xcopies = []
        for j in range(XBLK):
            xcopies.append(pltpu.async_copy(
                x3_hbm.at[wid * XBLK + j, 0],
                i0_v.at[pl.ds(j * 128, 128)], sem0))
            xcopies.append(pltpu.async_copy(
                x3_hbm.at[wid * XBLK + j, 1],
                i1_v.at[pl.ds(j * 128, 128)], sem1))
        for c in xcopies:
            c.wait()
        c0 = pltpu.async_copy(f0_hbm.at[i0_v], a0_v, sem0)
        c1 = pltpu.async_copy(f0_hbm.at[i1_v], a1_v, sem1)
        c2 = pltpu.async_copy(f1_hbm.at[i0_v], b0_v, sem2)
        c3 = pltpu.async_copy(f1_hbm.at[i1_v], b1_v, sem3)
        pltpu.sync_copy(deg_hbm.at[pl.ds(base, PAIRS_PER_W)], deg_v)
        c0.wait()
        c1.wait()
        c2.wait()
        c3.wait()

        def body(i, acc):
            s = pl.ds(i * LANES, LANES)
            d = jnp.abs(a0_v[s] - a1_v[s]) + jnp.abs(b0_v[s] - b1_v[s])
            return acc + deg_v[s] * d + jnp.exp(-d)

        acc = lax.fori_loop(0, CHUNKS, body, jnp.zeros((LANES,), jnp.float32))
        acc_v[...] = acc
        pltpu.sync_copy(acc_v, out_hbm.at[pl.ds(wid * LANES, LANES)])

    return k(f0, f1, x3, degrees)


def kernel(x, degrees, A, embs):
    f0, f1 = _tc_project(A.T, embs.T)
    x3 = x.reshape(128, 128, 2).transpose(0, 2, 1)
    partials = _sc_pair_loss(f0.reshape(-1), f1.reshape(-1), x3, degrees)
    return jnp.sum(partials)


# speedup vs baseline: 1.0093x; 1.0093x over previous
import functools

import jax
import jax.numpy as jnp
from jax import lax
from jax.experimental import pallas as pl
from jax.experimental.pallas import tpu as pltpu
from jax.experimental.pallas import tpu_sc as plsc

KMER_NUM = 1000000
DIM = 2
LATENT_DIM = 7
BATCH = 16384

NUM_CORES = 2
NUM_SUBCORES = 16
NUM_WORKERS = NUM_CORES * NUM_SUBCORES
PAIRS_PER_W = BATCH // NUM_WORKERS
LANES = 16
CHUNKS = PAIRS_PER_W // LANES

TC_BLOCK = 262144
TC_GRID = -(-KMER_NUM // TC_BLOCK)
F_ROWS = TC_GRID * TC_BLOCK // 128

LOG2E = 1.4426950408889634


def _tc_project_body(at_ref, et_ref, f0_ref, f1_ref):
    at = at_ref[...]
    for _ in range(2):
        at = jnp.exp(at - jnp.max(at, axis=1, keepdims=True))
        at = at / jnp.sum(at, axis=1, keepdims=True)
    e = et_ref[...]
    eb = (e * LOG2E).astype(jnp.bfloat16)
    ex = jnp.exp2(eb)
    mt = jnp.concatenate([at, jnp.ones((1, LATENT_DIM), jnp.float32)], axis=0)
    f3 = jax.lax.dot_general(
        mt.astype(jnp.bfloat16),
        ex,
        (((1,), (0,)), ((), ())),
        preferred_element_type=jnp.float32,
    )
    u0 = f3[0:1].reshape(TC_BLOCK // 128, 128)
    u1 = f3[1:2].reshape(TC_BLOCK // 128, 128)
    s = f3[2:3].reshape(TC_BLOCK // 128, 128)
    r = 1.0 / s
    f0_ref[...] = u0 * r
    f1_ref[...] = u1 * r


def _tc_project(At, embsT):
    return pl.pallas_call(
        _tc_project_body,
        grid=(TC_GRID,),
        in_specs=[
            pl.BlockSpec((DIM, LATENT_DIM), lambda i: (0, 0)),
            pl.BlockSpec((LATENT_DIM, TC_BLOCK), lambda i: (0, i)),
        ],
        out_specs=[
            pl.BlockSpec((TC_BLOCK // 128, 128), lambda i: (i, 0)),
            pl.BlockSpec((TC_BLOCK // 128, 128), lambda i: (i, 0)),
        ],
        out_shape=[
            jax.ShapeDtypeStruct((F_ROWS, 128), jnp.float32),
            jax.ShapeDtypeStruct((F_ROWS, 128), jnp.float32),
        ],
    )(At, embsT)


XBLK = PAIRS_PER_W // 128


def _sc_pair_loss(f0, f1, x3, degrees):
    mesh = plsc.VectorSubcoreMesh(core_axis_name="c", subcore_axis_name="s")

    @functools.partial(
        pl.kernel,
        mesh=mesh,
        compiler_params=pltpu.CompilerParams(use_tc_tiling_on_sc=False),
        out_type=jax.ShapeDtypeStruct((NUM_WORKERS * LANES,), jnp.float32),
        scratch_types=[
            pltpu.VMEM((PAIRS_PER_W,), jnp.int32),
            pltpu.VMEM((PAIRS_PER_W,), jnp.int32),
            pltpu.VMEM((PAIRS_PER_W,), jnp.float32),
            pltpu.VMEM((PAIRS_PER_W,), jnp.float32),
            pltpu.VMEM((PAIRS_PER_W,), jnp.float32),
            pltpu.VMEM((PAIRS_PER_W,), jnp.float32),
            pltpu.VMEM((PAIRS_PER_W,), jnp.float32),
            pltpu.VMEM((LANES,), jnp.float32),
            pltpu.SemaphoreType.DMA,
            pltpu.SemaphoreType.DMA,
            pltpu.SemaphoreType.DMA,
            pltpu.SemaphoreType.DMA,
        ],
    )
    def k(f0_hbm, f1_hbm, x3_hbm, deg_hbm, out_hbm,
          i0_v, i1_v, a0_v, a1_v, b0_v, b1_v, deg_v, acc_v,
          sem0, sem1, sem2, sem3):
        wid = lax.axis_index("s") * NUM_CORES + lax.axis_index("c")
        base = wid * PAIRS_PER_W
        xcopies = []
        for j in range(XBLK):
            xcopies.append(pltpu.async_copy(
                x3_hbm.at[wid * XBLK + j, 0],
                i0_v.at[pl.ds(j * 128, 128)], sem0))
            xcopies.append(pltpu.async_copy(
                x3_hbm.at[wid * XBLK + j, 1],
                i1_v.at[pl.ds(j * 128, 128)], sem1))
        for c in xcopies:
            c.wait()
        c0 = pltpu.async_copy(f0_hbm.at[i0_v], a0_v, sem0)
        c1 = pltpu.async_copy(f0_hbm.at[i1_v], a1_v, sem1)
        c2 = pltpu.async_copy(f1_hbm.at[i0_v], b0_v, sem2)
        c3 = pltpu.async_copy(f1_hbm.at[i1_v], b1_v, sem3)
        pltpu.sync_copy(deg_hbm.at[pl.ds(base, PAIRS_PER_W)], deg_v)
        c0.wait()
        c1.wait()
        c2.wait()
        c3.wait()

        def body(i, acc):
            s = pl.ds(i * LANES, LANES)
            d = jnp.abs(a0_v[s] - a1_v[s]) + jnp.abs(b0_v[s] - b1_v[s])
            return acc + deg_v[s] * d + jnp.exp(-d)

        acc = lax.fori_loop(0, CHUNKS, body, jnp.zeros((LANES,), jnp.float32))
        acc_v[...] = acc
        pltpu.sync_copy(acc_v, out_hbm.at[pl.ds(wid * LANES, LANES)])

    return k(f0, f1, x3, degrees)


def kernel(x, degrees, A, embs):
    f0, f1 = _tc_project(A.T, embs.T)
    x3 = x.reshape(128, 128, 2).transpose(0, 2, 1)
    partials = _sc_pair_loss(f0.reshape(-1), f1.reshape(-1), x3, degrees)
    return jnp.sum(partials)


# speedup vs baseline: 1.0102x; 1.0010x over previous
import functools

import jax
import jax.numpy as jnp
from jax import lax
from jax.experimental import pallas as pl
from jax.experimental.pallas import tpu as pltpu
from jax.experimental.pallas import tpu_sc as plsc

KMER_NUM = 1000000
DIM = 2
LATENT_DIM = 7
BATCH = 16384

NUM_CORES = 2
NUM_SUBCORES = 16
NUM_WORKERS = NUM_CORES * NUM_SUBCORES
PAIRS_PER_W = BATCH // NUM_WORKERS
LANES = 16
CHUNKS = PAIRS_PER_W // LANES

TC_BLOCK = 262144
TC_GRID = -(-KMER_NUM // TC_BLOCK)
F_ROWS = TC_GRID * TC_BLOCK // 128

LOG2E = 1.4426950408889634


def _tc_project_body(at_ref, et_ref, f0_ref, f1_ref):
    at = at_ref[...]
    for _ in range(2):
        at = jnp.exp(at - jnp.max(at, axis=1, keepdims=True))
        at = at / jnp.sum(at, axis=1, keepdims=True)
    e = et_ref[...]
    eb = (e * LOG2E).astype(jnp.bfloat16)
    ex = jnp.exp2(eb)
    mt = jnp.concatenate([at, jnp.ones((1, LATENT_DIM), jnp.float32)], axis=0)
    f3 = jax.lax.dot_general(
        mt.astype(jnp.bfloat16),
        ex,
        (((1,), (0,)), ((), ())),
        preferred_element_type=jnp.float32,
    )
    u0 = f3[0:1].reshape(TC_BLOCK // 128, 128)
    u1 = f3[1:2].reshape(TC_BLOCK // 128, 128)
    s = f3[2:3].reshape(TC_BLOCK // 128, 128)
    r = 1.0 / s
    f0_ref[...] = u0 * r
    f1_ref[...] = u1 * r


def _tc_project(At, embsT):
    return pl.pallas_call(
        _tc_project_body,
        grid=(TC_GRID,),
        in_specs=[
            pl.BlockSpec((DIM, LATENT_DIM), lambda i: (0, 0)),
            pl.BlockSpec((LATENT_DIM, TC_BLOCK), lambda i: (0, i)),
        ],
        out_specs=[
            pl.BlockSpec((TC_BLOCK // 128, 128), lambda i: (i, 0)),
            pl.BlockSpec((TC_BLOCK // 128, 128), lambda i: (i, 0)),
        ],
        out_shape=[
            jax.ShapeDtypeStruct((F_ROWS, 128), jnp.float32),
            jax.ShapeDtypeStruct((F_ROWS, 128), jnp.float32),
        ],
    )(At, embsT)


XBLK = PAIRS_PER_W // 128


def _sc_pair_loss(f0, f1, x3, degrees):
    mesh = plsc.VectorSubcoreMesh(core_axis_name="c", subcore_axis_name="s")

    @functools.partial(
        pl.kernel,
        mesh=mesh,
        compiler_params=pltpu.CompilerParams(use_tc_tiling_on_sc=False),
        out_type=jax.ShapeDtypeStruct((NUM_WORKERS * LANES,), jnp.float32),
        scratch_types=[
            pltpu.VMEM((PAIRS_PER_W,), jnp.int32),
            pltpu.VMEM((PAIRS_PER_W,), jnp.int32),
            pltpu.VMEM((PAIRS_PER_W,), jnp.float32),
            pltpu.VMEM((PAIRS_PER_W,), jnp.float32),
            pltpu.VMEM((PAIRS_PER_W,), jnp.float32),
            pltpu.VMEM((PAIRS_PER_W,), jnp.float32),
            pltpu.VMEM((PAIRS_PER_W,), jnp.float32),
            pltpu.VMEM((LANES,), jnp.float32),
            pltpu.SemaphoreType.DMA,
            pltpu.SemaphoreType.DMA,
            pltpu.SemaphoreType.DMA,
            pltpu.SemaphoreType.DMA,
        ],
    )
    def k(f0_hbm, f1_hbm, x3_hbm, deg_hbm, out_hbm,
          i0_v, i1_v, a0_v, a1_v, b0_v, b1_v, deg_v, acc_v,
          sem0, sem1, sem2, sem3):
        wid = lax.axis_index("s") * NUM_CORES + lax.axis_index("c")
        base = wid * PAIRS_PER_W
        x0copies = []
        x1copies = []
        for j in range(XBLK):
            x0copies.append(pltpu.async_copy(
                x3_hbm.at[wid * XBLK + j, 0],
                i0_v.at[pl.ds(j * 128, 128)], sem0))
            x1copies.append(pltpu.async_copy(
                x3_hbm.at[wid * XBLK + j, 1],
                i1_v.at[pl.ds(j * 128, 128)], sem1))
        for c in x0copies:
            c.wait()
        c0 = pltpu.async_copy(f0_hbm.at[i0_v], a0_v, sem0)
        c2 = pltpu.async_copy(f1_hbm.at[i0_v], b0_v, sem2)
        for c in x1copies:
            c.wait()
        c1 = pltpu.async_copy(f0_hbm.at[i1_v], a1_v, sem1)
        c3 = pltpu.async_copy(f1_hbm.at[i1_v], b1_v, sem3)
        pltpu.sync_copy(deg_hbm.at[pl.ds(base, PAIRS_PER_W)], deg_v)
        c0.wait()
        c1.wait()
        c2.wait()
        c3.wait()

        acc = jnp.zeros((LANES,), jnp.float32)
        for i in range(CHUNKS):
            s = pl.ds(i * LANES, LANES)
            d = jnp.abs(a0_v[s] - a1_v[s]) + jnp.abs(b0_v[s] - b1_v[s])
            acc = acc + deg_v[s] * d + jnp.exp(-d)
        acc_v[...] = acc
        pltpu.sync_copy(acc_v, out_hbm.at[pl.ds(wid * LANES, LANES)])

    return k(f0, f1, x3, degrees)


def kernel(x, degrees, A, embs):
    f0, f1 = _tc_project(A.T, embs.T)
    x3 = x.reshape(128, 128, 2).transpose(0, 2, 1)
    partials = _sc_pair_loss(f0.reshape(-1), f1.reshape(-1), x3, degrees)
    return jnp.sum(partials)
